# Initial kernel scaffold; baseline (speedup 1.0000x reference)
import jax
import jax.numpy as jnp
from jax.experimental import pallas as pl


def kernel(x, support, y, C_b_prime, Q, W, a, fc0_W, fc0_b, gcn_W):
    raise NotImplementedError("write your pallas kernel here")


# speedup vs baseline: 2.1644x; 2.1644x over previous
import functools

import jax
import jax.numpy as jnp
from jax.experimental import pallas as pl

_F32 = jnp.float32
_NEG = -9e15


def _dot(a, b):
    return jnp.dot(a, b, preferred_element_type=_F32)


def _dg(a, b, dims):
    return jax.lax.dot_general(a, b, (dims, ((), ())),
                               preferred_element_type=_F32)


def _prep_body(hid, kb, x_ref, w_ref, a_ref, fcw_ref, fcb_ref, cb_ref,
               wh_ref, wh1_ref, wh2r_ref, f0_ref, lsum_ref):
    xb = x_ref[...]
    whb = _dot(xb, w_ref[...])
    wh_ref[...] = whb
    a1 = a_ref[:hid, :]
    a2 = a_ref[hid:, :]
    wh1_ref[...] = _dot(whb, a1)
    wh2r_ref[...] = _dg(a2, whb, (((0,), (1,))))
    f0_ref[...] = _dg(whb, fcw_ref[...], (((1,), (1,)))) + fcb_ref[...]
    acc_log = None
    acc_s = None
    for k in range(kb):
        diff = xb - cb_ref[k:k + 1, :]
        d = jnp.sqrt(jnp.sum(diff * diff, axis=1, keepdims=True))
        s = jnp.exp(-d) + 1e-10
        ls = jnp.log(s)
        acc_log = ls if acc_log is None else acc_log + ls
        acc_s = s if acc_s is None else acc_s + s
    lsum_ref[...] = acc_log - kb * jnp.log(acc_s)


def _attn_logits(sup, wh1, wh2r):
    e = wh1 + wh2r
    e = jnp.where(e >= 0, e, 0.2 * e)
    return jnp.where(sup > 0, e, _NEG)


def _attn_body(c, sup_ref, wh1_ref, wh2r_ref, wh_ref, gcnw_ref, y_ref,
               z0_ref, xw_ref, cs_ref, cnt_ref):
    i = pl.program_id(0)
    logits = _attn_logits(sup_ref[...], wh1_ref[...], wh2r_ref[...])
    mx = jnp.max(logits, axis=1, keepdims=True)
    p = jnp.exp(logits - mx)
    sm = jnp.sum(p, axis=1, keepdims=True)
    hp = _dot(p, wh_ref[...]) / sm
    z0_ref[...] = hp
    gat = jnp.where(hp > 0, hp, jnp.exp(jnp.minimum(hp, 0.0)) - 1.0)
    xw_ref[...] = _dot(gat, gcnw_ref[...])
    yrow = y_ref[0]
    r = yrow.shape[1]
    oh = (jax.lax.broadcasted_iota(jnp.int32, (c, r), 0) == yrow)
    oh = oh.astype(_F32)
    csb = _dot(oh, hp)
    cntb = _dot(oh, jnp.ones((r, cs_ref.shape[1]), dtype=_F32))

    @pl.when(i == 0)
    def _init():
        cs_ref[...] = csb
        cnt_ref[...] = cntb

    @pl.when(i > 0)
    def _acc():
        cs_ref[...] += csb
        cnt_ref[...] += cntb


def _spread_body(sup_ref, xw_ref, o_ref):
    o_ref[...] = _dot(sup_ref[...], xw_ref[...])


def _final_body(c, n, sup_ref, z0_ref, wh1_ref, wh2r_ref, lsum_ref, y_ref,
                cs_ref, cnt_ref, q_ref, f0_ref, out_ref, res_ref):
    ca = cs_ref[...] / jnp.maximum(cnt_ref[...], 1.0)
    z0 = z0_ref[...]
    yrow = y_ref[0]
    r = yrow.shape[1]
    oh = (jax.lax.broadcasted_iota(jnp.int32, (c, r), 0) == yrow)
    oh = oh.astype(_F32)
    logq = jnp.log(q_ref[...])
    d_list, s_list = [], []
    ssum = None
    for k in range(c):
        diff = z0 - ca[k:k + 1, :]
        d = jnp.sqrt(jnp.sum(diff * diff, axis=1, keepdims=True))
        s = jnp.exp(-d) + 1e-10
        d_list.append(d)
        s_list.append(s)
        ssum = s if ssum is None else ssum + s
    lsum = lsum_ref[...]
    phi_list, db_list = [], []
    dsum = None
    for k in range(c):
        phi = s_list[k] / ssum
        sylq = _dg(oh, logq[:, k:k + 1], (((0,), (0,))))
        db = phi * lsum - sylq
        phi_list.append(phi)
        db_list.append(db)
        dsum = db if dsum is None else dsum + db
    csum = None
    vc = None
    for k in range(c):
        cmat = (dsum * phi_list[k] - db_list[k]) / (d_list[k] + 1e-12)
        csum = cmat if csum is None else csum + cmat
        term = cmat * ca[k:k + 1, :]
        vc = term if vc is None else vc + term
    v = csum * z0 - vc
    grad = _dg(v, f0_ref[...], (((1,), (1,)))) * (1.0 / n)
    logits = _attn_logits(sup_ref[...], wh1_ref[...], wh2r_ref[...])
    mx = jnp.max(logits, axis=1, keepdims=True)
    p = jnp.exp(logits - mx)
    sm = jnp.sum(p, axis=1, keepdims=True)
    b1 = p / sm - grad
    rn = jnp.sqrt(jnp.sum(b1 * b1, axis=1, keepdims=True))
    z1 = _dot(b1, out_ref[...]) / rn
    res_ref[...] = jnp.maximum(z1, 0.0)


def kernel(x, support, y, C_b_prime, Q, W, a, fc0_W, fc0_b, gcn_W):
    n, d_in = x.shape
    hid = W.shape[1]
    d_out = gcn_W.shape[1]
    c = Q.shape[0]
    kb = C_b_prime.shape[0]
    r = 128
    nblk = n // r
    y3 = y.astype(jnp.int32).reshape(nblk, 1, r)
    fcb = fc0_b.reshape(1, hid)

    row = lambda bs: pl.BlockSpec(bs, lambda i: (i, 0))
    full = lambda bs: pl.BlockSpec(bs, lambda i: (0, 0))

    wh, wh1, wh2r, f0, lsum = pl.pallas_call(
        functools.partial(_prep_body, hid, kb),
        grid=(nblk,),
        in_specs=[row((r, d_in)), full((d_in, hid)), full((2 * hid, 1)),
                  full((hid, hid)), full((1, hid)), full((kb, d_in))],
        out_specs=[row((r, hid)), row((r, 1)),
                   pl.BlockSpec((1, r), lambda i: (0, i)),
                   row((r, hid)), row((r, 1))],
        out_shape=[jax.ShapeDtypeStruct((n, hid), _F32),
                   jax.ShapeDtypeStruct((n, 1), _F32),
                   jax.ShapeDtypeStruct((1, n), _F32),
                   jax.ShapeDtypeStruct((n, hid), _F32),
                   jax.ShapeDtypeStruct((n, 1), _F32)],
    )(x, W, a, fc0_W, fcb, C_b_prime)

    z0, xw, cs, cnt = pl.pallas_call(
        functools.partial(_attn_body, c),
        grid=(nblk,),
        in_specs=[row((r, n)), row((r, 1)), full((1, n)), full((n, hid)),
                  full((hid, d_out)),
                  pl.BlockSpec((1, 1, r), lambda i: (i, 0, 0))],
        out_specs=[row((r, hid)), row((r, d_out)),
                   full((c, hid)), full((c, hid))],
        out_shape=[jax.ShapeDtypeStruct((n, hid), _F32),
                   jax.ShapeDtypeStruct((n, d_out), _F32),
                   jax.ShapeDtypeStruct((c, hid), _F32),
                   jax.ShapeDtypeStruct((c, hid), _F32)],
    )(support, wh1, wh2r, wh, gcn_W, y3)

    out = pl.pallas_call(
        _spread_body,
        grid=(nblk,),
        in_specs=[row((r, n)), full((n, d_out))],
        out_specs=row((r, d_out)),
        out_shape=jax.ShapeDtypeStruct((n, d_out), _F32),
    )(support, xw)

    res = pl.pallas_call(
        functools.partial(_final_body, c, n),
        grid=(nblk,),
        in_specs=[row((r, n)), row((r, hid)), row((r, 1)), full((1, n)),
                  row((r, 1)), pl.BlockSpec((1, 1, r), lambda i: (i, 0, 0)),
                  full((c, hid)), full((c, hid)), full((c, c)),
                  full((n, hid)), full((n, d_out))],
        out_specs=row((r, d_out)),
        out_shape=jax.ShapeDtypeStruct((n, d_out), _F32),
    )(support, z0, wh1, wh2r, lsum, y3, cs, cnt, Q, f0, out)
    return res


# speedup vs baseline: 2.5509x; 1.1786x over previous
import functools

import jax
import jax.numpy as jnp
from jax.experimental import pallas as pl
from jax.experimental.pallas import tpu as pltpu

_F32 = jnp.float32
_BF16 = jnp.bfloat16
_NEG = -9e15


def _dot(a, b):
    return jnp.dot(a, b, preferred_element_type=_F32)


def _dg(a, b, dims):
    return jax.lax.dot_general(a, b, (dims, ((), ())),
                               preferred_element_type=_F32)


def _prep_body(hid, kb, x_ref, w_ref, a_ref, fcw_ref, fcb_ref, cb_ref,
               wh_ref, wh1_ref, wh2r_ref, f0_ref, lsum_ref):
    xb = x_ref[...]
    whb = _dot(xb, w_ref[...])
    wh_ref[...] = whb
    a1 = a_ref[:hid, :]
    a2 = a_ref[hid:, :]
    wh1_ref[...] = _dot(whb, a1)
    wh2r_ref[...] = _dg(a2, whb, (((0,), (1,))))
    f0_ref[...] = _dg(whb, fcw_ref[...], (((1,), (1,)))) + fcb_ref[...]
    cb = cb_ref[...]
    xsq = jnp.sum(xb * xb, axis=1, keepdims=True)
    cbsq = _dg(jnp.ones((1, cb.shape[1]), _F32), cb * cb,
               (((1,), (1,))))
    cross = _dg(xb, cb, (((1,), (1,))))
    d2 = jnp.maximum(xsq - 2.0 * cross + cbsq, 0.0)
    d = jnp.sqrt(d2)
    s = jnp.exp(-d) + 1e-10
    lsum_ref[...] = (jnp.sum(jnp.log(s), axis=1, keepdims=True)
                     - kb * jnp.log(jnp.sum(s, axis=1, keepdims=True)))


def _fused_body(c, n, r, sup_ref, wh1_ref, wh2r_ref, wh_ref, gcnw_ref,
                y_ref, lsum_ref, q_ref, f0_ref, res_ref,
                z0_s, xw_s, out_s, cs_s, cnt_s, p_s, sm_s, supb_s):
    s = pl.program_id(0)
    i = pl.program_id(1)
    rows = pl.ds(i * r, r)

    @pl.when(s == 0)
    def _stage0():
        sup = sup_ref[...]
        supb_s[rows, :] = sup.astype(_BF16)
        e = wh1_ref[...] + wh2r_ref[...]
        e = jnp.where(e >= 0, e, 0.2 * e)
        logits = jnp.where(sup > 0, e, _NEG)
        mx = jnp.max(logits, axis=1, keepdims=True)
        p = jnp.exp(logits - mx)
        sm = jnp.sum(p, axis=1, keepdims=True)
        p_s[rows, :] = p.astype(_BF16)
        sm_s[rows, :] = sm
        hp = _dot(p, wh_ref[...]) / sm
        z0_s[rows, :] = hp
        gat = jnp.where(hp > 0, hp, jnp.exp(jnp.minimum(hp, 0.0)) - 1.0)
        xw_s[rows, :] = _dot(gat, gcnw_ref[...]).astype(_BF16)
        yrow = y_ref[0]
        oh = (jax.lax.broadcasted_iota(jnp.int32, (c, r), 0) == yrow)
        oh = oh.astype(_F32)
        csb = _dot(oh, hp)
        cntb = _dot(oh, jnp.ones((r, cs_s.shape[1]), dtype=_F32))

        @pl.when(i == 0)
        def _init():
            cs_s[...] = csb
            cnt_s[...] = cntb

        @pl.when(i > 0)
        def _acc():
            cs_s[...] += csb
            cnt_s[...] += cntb

    @pl.when(s == 1)
    def _stage1():
        out_s[rows, :] = _dot(supb_s[rows, :], xw_s[...])

    @pl.when(s == 2)
    def _stage2():
        ca = cs_s[...] / jnp.maximum(cnt_s[...], 1.0)
        z0 = z0_s[rows, :]
        yrow = y_ref[0]
        oh = (jax.lax.broadcasted_iota(jnp.int32, (c, r), 0) == yrow)
        oh = oh.astype(_F32)
        logq = jnp.log(q_ref[...])
        d_list, s_list = [], []
        ssum = None
        for k in range(c):
            diff = z0 - ca[k:k + 1, :]
            d = jnp.sqrt(jnp.sum(diff * diff, axis=1, keepdims=True))
            sk = jnp.exp(-d) + 1e-10
            d_list.append(d)
            s_list.append(sk)
            ssum = sk if ssum is None else ssum + sk
        lsum = lsum_ref[...]
        phi_list, db_list = [], []
        dsum = None
        for k in range(c):
            phi = s_list[k] / ssum
            sylq = _dg(oh, logq[:, k:k + 1], (((0,), (0,))))
            db = phi * lsum - sylq
            phi_list.append(phi)
            db_list.append(db)
            dsum = db if dsum is None else dsum + db
        csum = None
        vc = None
        for k in range(c):
            cmat = (dsum * phi_list[k] - db_list[k]) / (d_list[k] + 1e-12)
            csum = cmat if csum is None else csum + cmat
            term = cmat * ca[k:k + 1, :]
            vc = term if vc is None else vc + term
        v = csum * z0 - vc
        grad = _dg(v, f0_ref[...], (((1,), (1,)))) * (1.0 / n)
        att = p_s[rows, :].astype(_F32) / sm_s[rows, :]
        b1 = att - grad
        rn = jnp.sqrt(jnp.sum(b1 * b1, axis=1, keepdims=True))
        z1 = _dot(b1, out_s[...]) / rn
        res_ref[...] = jnp.maximum(z1, 0.0)


def kernel(x, support, y, C_b_prime, Q, W, a, fc0_W, fc0_b, gcn_W):
    n, d_in = x.shape
    hid = W.shape[1]
    d_out = gcn_W.shape[1]
    c = Q.shape[0]
    kb = C_b_prime.shape[0]
    r = 128
    nblk = n // r
    y3 = y.astype(jnp.int32).reshape(nblk, 1, r)
    fcb = fc0_b.reshape(1, hid)

    row = lambda bs: pl.BlockSpec(bs, lambda i: (i, 0))
    full = lambda bs: pl.BlockSpec(bs, lambda i: (0, 0))

    wh, wh1, wh2r, f0, lsum = pl.pallas_call(
        functools.partial(_prep_body, hid, kb),
        grid=(nblk,),
        in_specs=[row((r, d_in)), full((d_in, hid)), full((2 * hid, 1)),
                  full((hid, hid)), full((1, hid)), full((kb, d_in))],
        out_specs=[row((r, hid)), row((r, 1)),
                   pl.BlockSpec((1, r), lambda i: (0, i)),
                   row((r, hid)), row((r, 1))],
        out_shape=[jax.ShapeDtypeStruct((n, hid), _F32),
                   jax.ShapeDtypeStruct((n, 1), _F32),
                   jax.ShapeDtypeStruct((1, n), _F32),
                   jax.ShapeDtypeStruct((n, hid), _F32),
                   jax.ShapeDtypeStruct((n, 1), _F32)],
    )(x, W, a, fc0_W, fcb, C_b_prime)

    rowf = lambda bs: pl.BlockSpec(bs, lambda s, i: (i, 0))
    fullf = lambda bs: pl.BlockSpec(bs, lambda s, i: (0, 0))
    sup_spec = pl.BlockSpec(
        (r, n), lambda s, i: (jnp.where(s == 0, i, nblk - 1), 0))

    res = pl.pallas_call(
        functools.partial(_fused_body, c, n, r),
        grid=(3, nblk),
        in_specs=[sup_spec, rowf((r, 1)), fullf((1, n)), fullf((n, hid)),
                  fullf((hid, d_out)),
                  pl.BlockSpec((1, 1, r), lambda s, i: (i, 0, 0)),
                  rowf((r, 1)), fullf((c, c)), fullf((n, hid))],
        out_specs=pl.BlockSpec(
            (r, d_out), lambda s, i: (jnp.where(s == 2, i, 0), 0)),
        out_shape=jax.ShapeDtypeStruct((n, d_out), _F32),
        scratch_shapes=[pltpu.VMEM((n, hid), _F32),
                        pltpu.VMEM((n, d_out), _BF16),
                        pltpu.VMEM((n, d_out), _F32),
                        pltpu.VMEM((c, hid), _F32),
                        pltpu.VMEM((c, hid), _F32),
                        pltpu.VMEM((n, n), _BF16),
                        pltpu.VMEM((n, 1), _F32),
                        pltpu.VMEM((n, n), _BF16)],
    )(support, wh1, wh2r, wh, gcn_W, y3, lsum, Q, f0)
    return res
